# baseline (speedup 1.0000x reference)
import functools

import jax
import jax.numpy as jnp
from jax import lax
from jax.experimental import pallas as pl
from jax.experimental.pallas import tpu as pltpu
from jax.experimental.pallas import tpu_sc as plsc

NC = 2
NS = 16
NW = NC * NS
CHUNK = 128
WPAD = 48


def _sc_segment_sum(h_pad, src3, dst3, n_pad):
    n_chunks = src3.shape[1]
    rows_per_tile = n_pad // NS
    mesh = plsc.VectorSubcoreMesh(core_axis_name="c", subcore_axis_name="s")

    @functools.partial(
        pl.kernel,
        out_type=jax.ShapeDtypeStruct((NC, n_pad, WPAD), jnp.float32),
        mesh=mesh,
        scratch_types=[
            pltpu.VMEM((n_chunks, CHUNK), jnp.int32),
            pltpu.VMEM((n_chunks, CHUNK), jnp.int32),
            pltpu.VMEM((CHUNK, WPAD), jnp.float32),
            pltpu.VMEM((CHUNK, WPAD), jnp.float32),
            pltpu.VMEM((rows_per_tile, WPAD), jnp.float32),
            pltpu.VMEM_SHARED((n_pad, WPAD), jnp.float32),
            pltpu.VMEM_SHARED((n_pad, WPAD), jnp.float32),
            pltpu.SemaphoreType.DMA,
            pltpu.SemaphoreType.DMA,
        ],
        compiler_params=pltpu.CompilerParams(use_tc_tiling_on_sc=False),
    )
    def k(h_hbm, src_hbm, dst_hbm, out_hbm, src_v, dst_v, rows_a, rows_b,
          buf_v, acc_sh, h_sh, sem, sem_i):
        cid = lax.axis_index("c")
        sid = lax.axis_index("s")
        wid = sid * NC + cid
        r0 = sid * rows_per_tile

        ci = pltpu.async_copy(src_hbm.at[wid], src_v, sem_i)
        cj = pltpu.async_copy(dst_hbm.at[wid], dst_v, sem_i)
        ch = pltpu.async_copy(
            h_hbm.at[pl.ds(r0, rows_per_tile)],
            h_sh.at[pl.ds(r0, rows_per_tile)], sem)

        def zero_row(r, _):
            for j in range(WPAD // 16):
                buf_v[r, pl.ds(j * 16, 16)] = jnp.zeros((16,), jnp.float32)
            return 0
        lax.fori_loop(0, rows_per_tile, zero_row, 0)
        pltpu.sync_copy(buf_v, acc_sh.at[pl.ds(r0, rows_per_tile)])
        ci.wait()
        cj.wait()
        ch.wait()
        plsc.subcore_barrier()

        def body(j, _):
            ca = pltpu.async_copy(h_sh.at[src_v.at[j]], rows_a, sem)
            cb = pltpu.async_copy(h_sh.at[src_v.at[j + 1]], rows_b, sem_i)
            ca.wait()
            pltpu.sync_copy(rows_a, acc_sh.at[dst_v.at[j]], add=True)
            cb.wait()
            pltpu.sync_copy(rows_b, acc_sh.at[dst_v.at[j + 1]], add=True)
            return 0
        lax.fori_loop(0, n_chunks // 2, lambda i, c: body(2 * i, c), 0)

        plsc.subcore_barrier()
        pltpu.sync_copy(acc_sh.at[pl.ds(r0, rows_per_tile)], buf_v)
        pltpu.sync_copy(buf_v, out_hbm.at[cid, pl.ds(r0, rows_per_tile)])

    return k(h_pad, src3, dst3)


def _pad_cols(h):
    bm = h.shape[0]
    ones = jnp.ones((bm, 1), jnp.float32)
    zeros = jnp.zeros((bm, WPAD - h.shape[1] - 1), jnp.float32)
    return jnp.concatenate([h, ones, zeros], axis=1)


def _tc_init(x, w, n_pad, bm=1024):
    d = w.shape[1]

    def body(x_ref, w_ref, o_ref):
        h = jnp.maximum(
            jnp.dot(x_ref[...], w_ref[...],
                    preferred_element_type=jnp.float32), 0.0)
        o_ref[...] = _pad_cols(h)

    grid = (pl.cdiv(n_pad, bm),)
    return pl.pallas_call(
        body,
        grid=grid,
        in_specs=[
            pl.BlockSpec((bm, x.shape[1]), lambda i: (i, 0)),
            pl.BlockSpec((x.shape[1], d), lambda i: (0, 0)),
        ],
        out_specs=pl.BlockSpec((bm, WPAD), lambda i: (i, 0)),
        out_shape=jax.ShapeDtypeStruct((n_pad, WPAD), jnp.float32),
    )(x, w)


def _tc_layer(partials, w, n_pad, bm=1024):
    d = w.shape[0]

    def body(p_ref, w_ref, o_ref):
        s = p_ref[0] + p_ref[1]
        deg = s[:, d : d + 1]
        inv = 1.0 / jnp.maximum(deg, 1.0)
        agg = s[:, :d] * inv
        h = jnp.maximum(
            jnp.dot(agg, w_ref[...], preferred_element_type=jnp.float32), 0.0)
        o_ref[...] = _pad_cols(h)

    grid = (pl.cdiv(n_pad, bm),)
    return pl.pallas_call(
        body,
        grid=grid,
        in_specs=[
            pl.BlockSpec((NC, bm, WPAD), lambda i: (0, i, 0)),
            pl.BlockSpec((d, w.shape[1]), lambda i: (0, 0)),
        ],
        out_specs=pl.BlockSpec((bm, WPAD), lambda i: (i, 0)),
        out_shape=jax.ShapeDtypeStruct((n_pad, WPAD), jnp.float32),
    )(partials, w)


def _tc_final(h0p, h1p, partials, w, n, bm=1000):
    d = w.shape[0]

    def body(h0_ref, h1_ref, p_ref, w_ref, o_ref):
        s = p_ref[0] + p_ref[1]
        deg = s[:, d : d + 1]
        inv = 1.0 / jnp.maximum(deg, 1.0)
        agg = s[:, :d] * inv
        h2 = jnp.maximum(
            jnp.dot(agg, w_ref[...], preferred_element_type=jnp.float32), 0.0)
        o_ref[...] = jnp.concatenate(
            [h0_ref[:, :d], h1_ref[:, :d], h2], axis=1)

    grid = (pl.cdiv(n, bm),)
    return pl.pallas_call(
        body,
        grid=grid,
        in_specs=[
            pl.BlockSpec((bm, WPAD), lambda i: (i, 0)),
            pl.BlockSpec((bm, WPAD), lambda i: (i, 0)),
            pl.BlockSpec((NC, bm, WPAD), lambda i: (0, i, 0)),
            pl.BlockSpec((d, w.shape[1]), lambda i: (0, 0)),
        ],
        out_specs=pl.BlockSpec((bm, 3 * d), lambda i: (i, 0)),
        out_shape=jax.ShapeDtypeStruct((n, 3 * d), jnp.float32),
    )(h0p, h1p, partials, w)


@jax.jit
def kernel(x, edge_index, W_init, W1, W2):
    n = x.shape[0]
    e = edge_index.shape[1]
    n_pad = ((n + NS * 8) // (NS * 8)) * (NS * 8)
    ecb = NW * CHUNK * 2
    e_pad = ((e + ecb - 1) // ecb) * ecb

    pad = jnp.full((e_pad - e,), n, jnp.int32)
    src3 = jnp.concatenate([edge_index[0], pad]).reshape(NW, -1, CHUNK)
    dst3 = jnp.concatenate([edge_index[1], pad]).reshape(NW, -1, CHUNK)

    h0p = _tc_init(x, W_init, n_pad)
    p1 = _sc_segment_sum(h0p, src3, dst3, n_pad)
    h1p = _tc_layer(p1, W1, n_pad)
    p2 = _sc_segment_sum(h1p, src3, dst3, n_pad)
    return _tc_final(h0p, h1p, p2, W2, n)


# speedup vs baseline: 1.0061x; 1.0061x over previous
import functools

import jax
import jax.numpy as jnp
from jax import lax
from jax.experimental import pallas as pl
from jax.experimental.pallas import tpu as pltpu
from jax.experimental.pallas import tpu_sc as plsc

NC = 2
NS = 16
NW = NC * NS
CHUNK = 128
WPAD = 48


def _sc_segment_sum(h_pad, src3, dst3, zeros_slice, n_pad):
    n_chunks = src3.shape[1]
    rows_per_tile = n_pad // NS
    mesh = plsc.VectorSubcoreMesh(core_axis_name="c", subcore_axis_name="s")

    @functools.partial(
        pl.kernel,
        out_type=jax.ShapeDtypeStruct((NC, n_pad, WPAD), jnp.float32),
        mesh=mesh,
        scratch_types=[
            pltpu.VMEM((n_chunks, CHUNK), jnp.int32),
            pltpu.VMEM((n_chunks, CHUNK), jnp.int32),
            pltpu.VMEM((CHUNK, WPAD), jnp.float32),
            pltpu.VMEM((rows_per_tile, WPAD), jnp.float32),
            pltpu.VMEM_SHARED((n_pad, WPAD), jnp.float32),
            pltpu.VMEM_SHARED((n_pad, WPAD), jnp.float32),
            pltpu.SemaphoreType.DMA,
            pltpu.SemaphoreType.DMA,
        ],
        compiler_params=pltpu.CompilerParams(use_tc_tiling_on_sc=False),
    )
    def k(h_hbm, src_hbm, dst_hbm, z_hbm, out_hbm, src_v, dst_v, rows_v,
          buf_v, acc_sh, h_sh, sem, sem_i):
        cid = lax.axis_index("c")
        sid = lax.axis_index("s")
        wid = sid * NC + cid
        r0 = sid * rows_per_tile

        ci = pltpu.async_copy(src_hbm.at[wid], src_v, sem_i)
        cj = pltpu.async_copy(dst_hbm.at[wid], dst_v, sem_i)
        ch = pltpu.async_copy(
            h_hbm.at[pl.ds(r0, rows_per_tile)],
            h_sh.at[pl.ds(r0, rows_per_tile)], sem)
        cz = pltpu.async_copy(z_hbm, acc_sh.at[pl.ds(r0, rows_per_tile)], sem)
        ci.wait()
        cj.wait()
        ch.wait()
        cz.wait()
        plsc.subcore_barrier()

        def body(j, _):
            pltpu.async_copy(h_sh.at[src_v.at[j]], rows_v, sem).wait()
            pltpu.sync_copy(rows_v, acc_sh.at[dst_v.at[j]], add=True)
            return 0
        lax.fori_loop(0, n_chunks, body, 0)

        plsc.subcore_barrier()
        pltpu.sync_copy(acc_sh.at[pl.ds(r0, rows_per_tile)], buf_v)
        pltpu.sync_copy(buf_v, out_hbm.at[cid, pl.ds(r0, rows_per_tile)])

    return k(h_pad, src3, dst3, zeros_slice)


def _pad_cols(h):
    bm = h.shape[0]
    ones = jnp.ones((bm, 1), jnp.float32)
    zeros = jnp.zeros((bm, WPAD - h.shape[1] - 1), jnp.float32)
    return jnp.concatenate([h, ones, zeros], axis=1)


def _tc_init(x, w, n_pad, bm=1024):
    d = w.shape[1]

    def body(x_ref, w_ref, o_ref):
        h = jnp.maximum(
            jnp.dot(x_ref[...], w_ref[...],
                    preferred_element_type=jnp.float32), 0.0)
        o_ref[...] = _pad_cols(h)

    grid = (pl.cdiv(n_pad, bm),)
    return pl.pallas_call(
        body,
        grid=grid,
        in_specs=[
            pl.BlockSpec((bm, x.shape[1]), lambda i: (i, 0)),
            pl.BlockSpec((x.shape[1], d), lambda i: (0, 0)),
        ],
        out_specs=pl.BlockSpec((bm, WPAD), lambda i: (i, 0)),
        out_shape=jax.ShapeDtypeStruct((n_pad, WPAD), jnp.float32),
    )(x, w)


def _tc_layer(partials, w, n_pad, bm=1024):
    d = w.shape[0]

    def body(p_ref, w_ref, o_ref):
        s = p_ref[0] + p_ref[1]
        deg = s[:, d : d + 1]
        inv = 1.0 / jnp.maximum(deg, 1.0)
        agg = s[:, :d] * inv
        h = jnp.maximum(
            jnp.dot(agg, w_ref[...], preferred_element_type=jnp.float32), 0.0)
        o_ref[...] = _pad_cols(h)

    grid = (pl.cdiv(n_pad, bm),)
    return pl.pallas_call(
        body,
        grid=grid,
        in_specs=[
            pl.BlockSpec((NC, bm, WPAD), lambda i: (0, i, 0)),
            pl.BlockSpec((d, w.shape[1]), lambda i: (0, 0)),
        ],
        out_specs=pl.BlockSpec((bm, WPAD), lambda i: (i, 0)),
        out_shape=jax.ShapeDtypeStruct((n_pad, WPAD), jnp.float32),
    )(partials, w)


def _tc_final(h0p, h1p, partials, w, n, bm=1000):
    d = w.shape[0]

    def body(h0_ref, h1_ref, p_ref, w_ref, o_ref):
        s = p_ref[0] + p_ref[1]
        deg = s[:, d : d + 1]
        inv = 1.0 / jnp.maximum(deg, 1.0)
        agg = s[:, :d] * inv
        h2 = jnp.maximum(
            jnp.dot(agg, w_ref[...], preferred_element_type=jnp.float32), 0.0)
        o_ref[...] = jnp.concatenate(
            [h0_ref[:, :d], h1_ref[:, :d], h2], axis=1)

    grid = (pl.cdiv(n, bm),)
    return pl.pallas_call(
        body,
        grid=grid,
        in_specs=[
            pl.BlockSpec((bm, WPAD), lambda i: (i, 0)),
            pl.BlockSpec((bm, WPAD), lambda i: (i, 0)),
            pl.BlockSpec((NC, bm, WPAD), lambda i: (0, i, 0)),
            pl.BlockSpec((d, w.shape[1]), lambda i: (0, 0)),
        ],
        out_specs=pl.BlockSpec((bm, 3 * d), lambda i: (i, 0)),
        out_shape=jax.ShapeDtypeStruct((n, 3 * d), jnp.float32),
    )(h0p, h1p, partials, w)


@jax.jit
def kernel(x, edge_index, W_init, W1, W2):
    n = x.shape[0]
    e = edge_index.shape[1]
    n_pad = ((n + NS * 8) // (NS * 8)) * (NS * 8)
    ecb = NW * CHUNK
    e_pad = ((e + ecb - 1) // ecb) * ecb

    pad = jnp.full((e_pad - e,), n, jnp.int32)
    src3 = jnp.concatenate([edge_index[0], pad]).reshape(NW, -1, CHUNK)
    dst3 = jnp.concatenate([edge_index[1], pad]).reshape(NW, -1, CHUNK)

    zeros_slice = jnp.zeros((n_pad // NS, WPAD), jnp.float32)
    h0p = _tc_init(x, W_init, n_pad)
    p1 = _sc_segment_sum(h0p, src3, dst3, zeros_slice, n_pad)
    h1p = _tc_layer(p1, W1, n_pad)
    p2 = _sc_segment_sum(h1p, src3, dst3, zeros_slice, n_pad)
    return _tc_final(h0p, h1p, p2, W2, n)
